# baseline (speedup 1.0000x reference)
import functools
import jax
import jax.numpy as jnp
from jax import lax
from jax.experimental import pallas as pl
from jax.experimental.pallas import tpu as pltpu
from jax.experimental.pallas import tpu_sc as plsc

NC = 2
NS = 16
NW = NC * NS
CH = 128
NB = 2


def _deg_body(per_worker, nfull, tail, dst_hbm, zeros_hbm, rowid_hbm, out_hbm,
              idxs, tailv, rowid_v, hist, acc):
    cid = lax.axis_index("c")
    sid = lax.axis_index("s")
    wid = sid * NC + cid
    e0 = wid * per_worker

    pltpu.sync_copy(zeros_hbm, hist)
    pltpu.sync_copy(rowid_hbm, rowid_v)
    pltpu.sync_copy(dst_hbm.at[pl.ds(e0, nfull * CH)], idxs)
    if tail:
        pltpu.sync_copy(dst_hbm.at[pl.ds(e0 + nfull * CH, tail)], tailv)

    @pl.when(sid == 0)
    def _():
        pltpu.sync_copy(zeros_hbm, acc)
    plsc.subcore_barrier()

    ones = jnp.ones((16,), jnp.float32)

    def chunk(i, carry):
        iv = idxs[pl.ds(i * 16, 16)]
        plsc.addupdate_scatter(hist, [iv >> 7, iv & 127], ones)
        return carry

    lax.fori_loop(0, nfull * CH // 16, chunk, 0)
    if tail:
        for j in range(tail // 16):
            iv = tailv[pl.ds(j * 16, 16)]
            plsc.addupdate_scatter(hist, [iv >> 7, iv & 127], ones)
    pltpu.sync_copy(hist, acc.at[rowid_v.at[0]], add=True)
    plsc.subcore_barrier()

    @pl.when(sid == 0)
    def _():
        pltpu.sync_copy(acc, out_hbm.at[cid])


def _agg_body(per_worker, nfull, tail, n, np_rows, rows_per_tile, y_hbm,
              src_hbm, dst_hbm, zeros_hbm, out_hbm, idxb, rows, isem, gsem,
              ssem, acc):
    cid = lax.axis_index("c")
    sid = lax.axis_index("s")
    wid = sid * NC + cid
    base = sid * rows_per_tile
    nblk = rows_per_tile // CH
    nrounds = nfull // NB
    e0 = wid * per_worker

    def load_slot(slot, r, copy):
        res = []
        for b in range(NB):
            off = e0 + (r * NB + b) * CH
            res.append(copy(src_hbm.at[pl.ds(off, CH)], idxb.at[slot, b, 0]))
            res.append(copy(dst_hbm.at[pl.ds(off, CH)], idxb.at[slot, b, 1]))
        return res

    load_slot(0, 0, pltpu.sync_copy)
    load_slot(1, 1, lambda s, d: pltpu.async_copy(s, d, isem))
    zinit = [pltpu.async_copy(zeros_hbm.at[pl.ds(base + j * CH, CH)],
                              acc.at[pl.ds(base + j * CH, CH)], gsem)
             for j in range(nblk)]
    for z in zinit:
        z.wait()
    plsc.subcore_barrier()

    def step(t, carry):
        p = lax.rem(t, 4)
        gathers = []
        for b in range(NB):
            @pl.when(t > 0)
            def _():
                pltpu.make_async_copy(
                    rows.at[b], acc.at[idxb.at[0, 0, 1]], ssem).wait()
            gathers.append(pltpu.async_copy(
                y_hbm.at[idxb.at[p, b, 0]], rows.at[b], gsem))

        @pl.when(t + 2 < nrounds)
        def _():
            load_slot(lax.rem(t + 2, 4), t + 2,
                      lambda s, d: pltpu.async_copy(s, d, isem))

        for b in range(NB):
            gathers[b].wait()
            pltpu.async_copy(rows.at[b], acc.at[idxb.at[p, b, 1]],
                             ssem, add=True)

        @pl.when(t + 1 < nrounds)
        def _():
            for _i in range(2 * NB):
                pltpu.make_async_copy(src_hbm.at[pl.ds(e0, CH)],
                                      idxb.at[0, 0, 0], isem).wait()
        return carry

    lax.fori_loop(0, nrounds, step, 0)
    for b in range(NB):
        pltpu.make_async_copy(rows.at[b], acc.at[idxb.at[0, 0, 1]],
                              ssem).wait()

    if tail:
        pltpu.sync_copy(src_hbm.at[pl.ds(e0 + nfull * CH, tail)],
                        idxb.at[0, 0, 0, pl.ds(0, tail)])
        pltpu.sync_copy(dst_hbm.at[pl.ds(e0 + nfull * CH, tail)],
                        idxb.at[0, 0, 1, pl.ds(0, tail)])
        lanes = lax.iota(jnp.int32, 16)
        for j in range(tail // 16, CH // 16):
            idxb[0, 0, 0, pl.ds(j * 16, 16)] = lanes + j * 16
            idxb[0, 0, 1, pl.ds(j * 16, 16)] = n + (lanes + j * 16) % (
                np_rows - n)
        pltpu.async_copy(y_hbm.at[idxb.at[0, 0, 0]], rows.at[0], gsem).wait()
        pltpu.sync_copy(rows.at[0], acc.at[idxb.at[0, 0, 1]], add=True)

    plsc.subcore_barrier()
    wb = [pltpu.async_copy(acc.at[pl.ds(base + j * CH, CH)],
                           out_hbm.at[cid, pl.ds(base + j * CH, CH)], gsem)
          for j in range(nblk)]
    for w in wb:
        w.wait()


def _scale_body(feat_ref, degp_ref, y_ref, dinv_ref):
    deg = degp_ref[0] + degp_ref[1]
    dinv = lax.rsqrt(jnp.maximum(deg, 1.0))
    dinv_ref[...] = dinv
    y_ref[...] = feat_ref[...] * dinv


def _x1_body(part_ref, dinv_ref, x1_ref, y1_ref):
    dinv = dinv_ref[...]
    x1 = -(part_ref[0] + part_ref[1]) * dinv
    x1_ref[...] = x1
    y1_ref[...] = x1 * dinv


def _final_body(feat_ref, x1_ref, part_ref, dinv_ref, out_ref):
    x0 = feat_ref[...]
    x2 = -2.0 * (part_ref[0] + part_ref[1]) * dinv_ref[...] - x0
    zero = jnp.float32(0.0)
    out_ref[:, 0:128] = jnp.maximum(x0, zero)
    out_ref[:, 128:256] = jnp.maximum(x1_ref[...], zero)
    out_ref[:, 256:384] = jnp.maximum(x2, zero)


def kernel(feat, edge_index):
    n, d = feat.shape
    e = edge_index.shape[1]
    assert d == 128

    np_rows = ((n + NS * CH - 1) // (NS * CH)) * (NS * CH)
    rows_per_tile = np_rows // NS
    assert e % NW == 0
    per_worker = e // NW
    nfull = (per_worker // (CH * NB)) * NB
    tail = per_worker - nfull * CH
    assert tail % 16 == 0 and tail < CH

    zeros_w = jnp.zeros((np_rows, d), jnp.float32)
    zeros80 = jnp.zeros((np_rows // CH, d), jnp.float32)
    rowids = jnp.arange(np_rows // CH, dtype=jnp.int32).reshape(1, -1)

    mesh = plsc.VectorSubcoreMesh(core_axis_name="c", subcore_axis_name="s")

    deg_k = pl.kernel(
        functools.partial(_deg_body, per_worker, nfull, tail),
        out_type=jax.ShapeDtypeStruct((NC, np_rows // CH, d), jnp.float32),
        mesh=mesh,
        scratch_types=[
            pltpu.VMEM((nfull * CH,), jnp.int32),
            pltpu.VMEM((max(tail, 16),), jnp.int32),
            pltpu.VMEM((1, np_rows // CH), jnp.int32),
            pltpu.VMEM((np_rows // CH, d), jnp.float32),
            pltpu.VMEM_SHARED((np_rows // CH, d), jnp.float32),
        ],
        compiler_params=pltpu.CompilerParams(needs_layout_passes=False),
    )

    agg_k = pl.kernel(
        functools.partial(_agg_body, per_worker, nfull, tail, n, np_rows,
                          rows_per_tile),
        out_type=jax.ShapeDtypeStruct((NC, np_rows, d), jnp.float32),
        mesh=mesh,
        scratch_types=[
            pltpu.VMEM((4, NB, 2, CH), jnp.int32),
            pltpu.VMEM((NB, CH, d), jnp.float32),
            pltpu.SemaphoreType.DMA,
            pltpu.SemaphoreType.DMA,
            pltpu.SemaphoreType.DMA,
            pltpu.VMEM_SHARED((np_rows, d), jnp.float32),
        ],
    )

    blk = 2000
    grid = (n // blk,)
    fspec = pl.BlockSpec((blk, d), lambda i: (i, 0))
    pspec = pl.BlockSpec((NC, blk, d), lambda i: (0, i, 0))
    dspec = pl.BlockSpec((NC, blk, 1), lambda i: (0, i, 0))
    vspec = pl.BlockSpec((blk, 1), lambda i: (i, 0))

    src = edge_index[0]
    dst = edge_index[1]
    deg_part = deg_k(dst, zeros80, rowids).reshape(NC, np_rows, 1)

    y0, dinv_b = pl.pallas_call(
        _scale_body,
        grid=grid,
        in_specs=[fspec, dspec],
        out_specs=[fspec, vspec],
        out_shape=[jax.ShapeDtypeStruct((n, d), jnp.float32),
                   jax.ShapeDtypeStruct((n, 1), jnp.float32)],
    )(feat, deg_part)

    part0 = agg_k(y0, src, dst, zeros_w)

    x1, y1 = pl.pallas_call(
        _x1_body,
        grid=grid,
        in_specs=[pspec, vspec],
        out_specs=[fspec, fspec],
        out_shape=[jax.ShapeDtypeStruct((n, d), jnp.float32)] * 2,
    )(part0, dinv_b)

    part1 = agg_k(y1, src, dst, zeros_w)

    out = pl.pallas_call(
        _final_body,
        grid=grid,
        in_specs=[fspec, fspec, pspec, vspec],
        out_specs=pl.BlockSpec((blk, 3 * d), lambda i: (i, 0)),
        out_shape=jax.ShapeDtypeStruct((n, 3 * d), jnp.float32),
    )(feat, x1, part1, dinv_b)

    return out


# speedup vs baseline: 1.0142x; 1.0142x over previous
import functools
import jax
import jax.numpy as jnp
from jax import lax
from jax.experimental import pallas as pl
from jax.experimental.pallas import tpu as pltpu
from jax.experimental.pallas import tpu_sc as plsc

NC = 2
NS = 16
NW = NC * NS
CH = 128
NB = 2


def _deg_body(nchunk, dst_hbm, zeros_hbm, rowid_hbm, out_hbm,
              idxs, rowid_v, hist, acc):
    cid = lax.axis_index("c")
    sid = lax.axis_index("s")
    wid = sid * NC + cid

    pltpu.sync_copy(zeros_hbm, hist)
    pltpu.sync_copy(rowid_hbm, rowid_v)
    pltpu.sync_copy(dst_hbm.at[pl.ds(wid * nchunk, nchunk)], idxs)

    @pl.when(sid == 0)
    def _():
        pltpu.sync_copy(zeros_hbm, acc)
    plsc.subcore_barrier()

    ones = jnp.ones((16,), jnp.float32)

    def chunk(k, carry):
        for j in range(CH // 16):
            iv = idxs[k, pl.ds(j * 16, 16)]
            plsc.addupdate_scatter(hist, [iv >> 7, iv & 127], ones)
        return carry

    lax.fori_loop(0, nchunk, chunk, 0)
    pltpu.sync_copy(hist, acc.at[rowid_v.at[0]], add=True)
    plsc.subcore_barrier()

    @pl.when(sid == 0)
    def _():
        pltpu.sync_copy(acc, out_hbm.at[cid])


def _agg_body(nchunk, rows_per_tile, y_hbm, ei_hbm, zeros_hbm,
              out_hbm, idxb, rows, isem, gsem, ssem, acc):
    cid = lax.axis_index("c")
    sid = lax.axis_index("s")
    wid = sid * NC + cid
    base = sid * rows_per_tile
    nblk = rows_per_tile // CH
    nrounds = nchunk // NB
    c0 = wid * nchunk

    pltpu.sync_copy(ei_hbm.at[pl.ds(c0, NB)], idxb.at[0])
    pltpu.async_copy(ei_hbm.at[pl.ds(c0 + NB, NB)], idxb.at[1], isem)
    zinit = [pltpu.async_copy(zeros_hbm.at[pl.ds(base + j * CH, CH)],
                              acc.at[pl.ds(base + j * CH, CH)], gsem)
             for j in range(nblk)]
    for z in zinit:
        z.wait()
    plsc.subcore_barrier()

    def step(t, carry):
        p = lax.rem(t, 4)
        gathers = []
        for b in range(NB):
            @pl.when(t > 0)
            def _():
                pltpu.make_async_copy(
                    rows.at[b], acc.at[idxb.at[0, 0, 1]], ssem).wait()
            gathers.append(pltpu.async_copy(
                y_hbm.at[idxb.at[p, b, 0]], rows.at[b], gsem))

        @pl.when(t + 2 < nrounds)
        def _():
            pltpu.async_copy(ei_hbm.at[pl.ds(c0 + (t + 2) * NB, NB)],
                             idxb.at[lax.rem(t + 2, 4)], isem)

        for b in range(NB):
            gathers[b].wait()
            pltpu.async_copy(rows.at[b], acc.at[idxb.at[p, b, 1]],
                             ssem, add=True)

        @pl.when(t + 1 < nrounds)
        def _():
            pltpu.make_async_copy(ei_hbm.at[pl.ds(c0, NB)], idxb.at[0],
                                  isem).wait()
        return carry

    lax.fori_loop(0, nrounds, step, 0)
    for b in range(NB):
        pltpu.make_async_copy(rows.at[b], acc.at[idxb.at[0, 0, 1]],
                              ssem).wait()
    plsc.subcore_barrier()
    wb = [pltpu.async_copy(acc.at[pl.ds(base + j * CH, CH)],
                           out_hbm.at[cid, pl.ds(base + j * CH, CH)], gsem)
          for j in range(nblk)]
    for w in wb:
        w.wait()


def _scale_body(feat_ref, degp_ref, y_ref, dinv_ref):
    deg = degp_ref[0] + degp_ref[1]
    dinv = lax.rsqrt(jnp.maximum(deg, 1.0))
    dinv_ref[...] = dinv
    y_ref[...] = feat_ref[...] * dinv


def _x1_body(part_ref, dinv_ref, x1_ref, y1_ref):
    dinv = dinv_ref[...]
    x1 = -(part_ref[0] + part_ref[1]) * dinv
    x1_ref[...] = x1
    y1_ref[...] = x1 * dinv


def _final_body(feat_ref, x1_ref, part_ref, dinv_ref, out_ref):
    x0 = feat_ref[...]
    x2 = -2.0 * (part_ref[0] + part_ref[1]) * dinv_ref[...] - x0
    zero = jnp.float32(0.0)
    out_ref[:, 0:128] = jnp.maximum(x0, zero)
    out_ref[:, 128:256] = jnp.maximum(x1_ref[...], zero)
    out_ref[:, 256:384] = jnp.maximum(x2, zero)


def kernel(feat, edge_index):
    n, d = feat.shape
    e = edge_index.shape[1]
    assert d == 128

    np_rows = ((n + NS * CH - 1) // (NS * CH)) * (NS * CH)
    rows_per_tile = np_rows // NS
    ew = ((e + NW * CH * NB - 1) // (NW * CH * NB)) * CH * NB
    ep = ew * NW
    nchunk = ew // CH

    src = edge_index[0]
    dst = edge_index[1]
    npad = ep - e
    pad_ids = jnp.arange(npad, dtype=jnp.int32)
    src_p = jnp.concatenate([src, (pad_ids * 997) % n]).reshape(ep // CH, CH)
    dst_p = jnp.concatenate([dst, n + pad_ids % (np_rows - n)]).reshape(
        ep // CH, CH)
    ei = jnp.stack([src_p, dst_p], axis=1)

    zeros_w = jnp.zeros((np_rows, d), jnp.float32)
    zeros80 = jnp.zeros((np_rows // CH, d), jnp.float32)
    rowids = jnp.arange(np_rows // CH, dtype=jnp.int32).reshape(1, -1)

    mesh = plsc.VectorSubcoreMesh(core_axis_name="c", subcore_axis_name="s")

    deg_k = pl.kernel(
        functools.partial(_deg_body, nchunk),
        out_type=jax.ShapeDtypeStruct((NC, np_rows // CH, d), jnp.float32),
        mesh=mesh,
        scratch_types=[
            pltpu.VMEM((nchunk, CH), jnp.int32),
            pltpu.VMEM((1, np_rows // CH), jnp.int32),
            pltpu.VMEM((np_rows // CH, d), jnp.float32),
            pltpu.VMEM_SHARED((np_rows // CH, d), jnp.float32),
        ],
        compiler_params=pltpu.CompilerParams(needs_layout_passes=False),
    )

    agg_k = pl.kernel(
        functools.partial(_agg_body, nchunk, rows_per_tile),
        out_type=jax.ShapeDtypeStruct((NC, np_rows, d), jnp.float32),
        mesh=mesh,
        scratch_types=[
            pltpu.VMEM((4, NB, 2, CH), jnp.int32),
            pltpu.VMEM((NB, CH, d), jnp.float32),
            pltpu.SemaphoreType.DMA,
            pltpu.SemaphoreType.DMA,
            pltpu.SemaphoreType.DMA,
            pltpu.VMEM_SHARED((np_rows, d), jnp.float32),
        ],
    )

    blk = 2000
    grid = (n // blk,)
    fspec = pl.BlockSpec((blk, d), lambda i: (i, 0))
    pspec = pl.BlockSpec((NC, blk, d), lambda i: (0, i, 0))
    dspec = pl.BlockSpec((NC, blk, 1), lambda i: (0, i, 0))
    vspec = pl.BlockSpec((blk, 1), lambda i: (i, 0))

    deg_part = deg_k(dst_p, zeros80, rowids).reshape(NC, np_rows, 1)

    y0, dinv_b = pl.pallas_call(
        _scale_body,
        grid=grid,
        in_specs=[fspec, dspec],
        out_specs=[fspec, vspec],
        out_shape=[jax.ShapeDtypeStruct((n, d), jnp.float32),
                   jax.ShapeDtypeStruct((n, 1), jnp.float32)],
    )(feat, deg_part)

    part0 = agg_k(y0, ei, zeros_w)

    x1, y1 = pl.pallas_call(
        _x1_body,
        grid=grid,
        in_specs=[pspec, vspec],
        out_specs=[fspec, fspec],
        out_shape=[jax.ShapeDtypeStruct((n, d), jnp.float32)] * 2,
    )(part0, dinv_b)

    part1 = agg_k(y1, ei, zeros_w)

    out = pl.pallas_call(
        _final_body,
        grid=grid,
        in_specs=[fspec, fspec, pspec, vspec],
        out_specs=pl.BlockSpec((blk, 3 * d), lambda i: (i, 0)),
        out_shape=jax.ShapeDtypeStruct((n, 3 * d), jnp.float32),
    )(feat, x1, part1, dinv_b)

    return out
